# Initial kernel scaffold; baseline (speedup 1.0000x reference)
import jax
import jax.numpy as jnp
from jax.experimental import pallas as pl


def kernel(hidden_states, W_key, W_value, W_expert_states_flat, W_expert_query, W_gate_proj, W_up_proj, W_down_proj):
    raise NotImplementedError("write your pallas kernel here")


# speedup vs baseline: 2.0526x; 2.0526x over previous
import jax
import jax.numpy as jnp
from jax import lax
from jax.experimental import pallas as pl
from jax.experimental.pallas import tpu as pltpu

_N = 2048
_H = 768
_E = 8
_F = 1024
_EH = _E * _H

_BF = jnp.bfloat16
_F32 = jnp.float32


def _mm(a, b):
    return lax.dot_general(a, b, (((1,), (0,)), ((), ())),
                           preferred_element_type=_F32)


def _mmT(a, b):
    return lax.dot_general(a, b, (((1,), (1,)), ((), ())),
                           preferred_element_type=_F32)


def _efs_kernel(xb_ref, wesf_ref, efs_ref):
    efs_ref[...] = _mm(xb_ref[...], wesf_ref[...]).astype(_BF)


def _efq_kernel(efs_ref, weq_ref, efq_ref):
    efq_ref[...] = _mm(efs_ref[...], weq_ref[...]).astype(_BF)


def _prep_kernel(xb_ref, wk_ref, wv_ref, kb_ref, vb_ref, cnull_ref):
    xb = xb_ref[...]
    kb_ref[...] = _mm(xb, wk_ref[...]).astype(_BF)
    vb = _mm(xb, wv_ref[...]).astype(_BF)
    vb_ref[...] = vb
    cnull_ref[...] = (jnp.sum(vb.astype(_F32)).reshape(1, 1)
                      * _F32(1.0 / (2 * _N)))


def _attn_kernel(q_ref, kb_ref, vb_ref, cnull_ref, w_ref, coeff_ref):
    kb = kb_ref[...]
    vb = vb_ref[...]
    sqrt_h = jnp.sqrt(_F32(_H))
    cols = []
    for e in range(_E):
        qe = q_ref[:, e * _H:(e + 1) * _H]
        l = _mmT(qe, kb) / sqrt_h
        m = jnp.maximum(jnp.max(l, axis=1, keepdims=True), 0.0)
        ex = jnp.exp(l - m)
        z = jnp.sum(ex, axis=1, keepdims=True) + _F32(_N) * jnp.exp(-m)
        p = (ex / z).astype(_BF)
        aw = _mm(p, vb)
        cols.append(jnp.sum(aw, axis=1, keepdims=True))
    a_real = jnp.concatenate(cols, axis=1)
    cn = cnull_ref[0, 0]
    w = a_real / (a_real + cn)
    w_ref[...] = w
    coeff_ref[...] = jnp.where((a_real - cn) > 0, w, 0.0)


def _mlp_kernel(xb_ref, wg_ref, wu_ref, wd_ref, coeff_ref, out_ref):
    e = pl.program_id(1)
    xb = xb_ref[...]
    g = _mm(xb, wg_ref[0])
    g = g * jax.nn.sigmoid(g)
    u = _mm(xb, wu_ref[0])
    y = _mm((g * u).astype(_BF), wd_ref[0])
    onehot = jax.lax.broadcasted_iota(jnp.int32, (1, _E), 1) == e
    ce = jnp.sum(jnp.where(onehot, coeff_ref[...], 0.0), axis=1, keepdims=True)
    contrib = y * ce

    @pl.when(e == 0)
    def _():
        out_ref[...] = contrib

    @pl.when(e != 0)
    def _():
        out_ref[...] += contrib


def kernel(hidden_states, W_key, W_value, W_expert_states_flat, W_expert_query,
           W_gate_proj, W_up_proj, W_down_proj):
    b, s, h = hidden_states.shape
    x = hidden_states.reshape(_N, _H)

    xb = x.astype(_BF)
    wesf_b = W_expert_states_flat.astype(_BF)
    weq_b = W_expert_query.astype(_BF)
    wk_b = W_key.astype(_BF)
    wv_b = W_value.astype(_BF)
    wg_b = W_gate_proj.astype(_BF)
    wu_b = W_up_proj.astype(_BF)
    wd_b = W_down_proj.astype(_BF)

    jb = 512
    efs_b = pl.pallas_call(
        _efs_kernel,
        grid=(_EH // jb,),
        in_specs=[
            pl.BlockSpec((_N, _H), lambda j: (0, 0)),
            pl.BlockSpec((_H, jb), lambda j: (0, j)),
        ],
        out_specs=pl.BlockSpec((_N, jb), lambda j: (0, j)),
        out_shape=jax.ShapeDtypeStruct((_N, _EH), _BF),
        compiler_params=pltpu.CompilerParams(
            dimension_semantics=("arbitrary",)),
    )(xb, wesf_b)

    efq_b = pl.pallas_call(
        _efq_kernel,
        grid=(_EH // jb,),
        in_specs=[
            pl.BlockSpec((_N, _EH), lambda j: (0, 0)),
            pl.BlockSpec((_EH, jb), lambda j: (0, j)),
        ],
        out_specs=pl.BlockSpec((_N, jb), lambda j: (0, j)),
        out_shape=jax.ShapeDtypeStruct((_N, _EH), _BF),
        compiler_params=pltpu.CompilerParams(
            dimension_semantics=("arbitrary",)),
    )(efs_b, weq_b)

    kb, vb, cnull = pl.pallas_call(
        _prep_kernel,
        out_shape=(
            jax.ShapeDtypeStruct((_N, _H), _BF),
            jax.ShapeDtypeStruct((_N, _H), _BF),
            jax.ShapeDtypeStruct((1, 1), _F32),
        ),
    )(xb, wk_b, wv_b)

    tb = 256
    w, coeff = pl.pallas_call(
        _attn_kernel,
        grid=(_N // tb,),
        in_specs=[
            pl.BlockSpec((tb, _EH), lambda t: (t, 0)),
            pl.BlockSpec((_N, _H), lambda t: (0, 0)),
            pl.BlockSpec((_N, _H), lambda t: (0, 0)),
            pl.BlockSpec((1, 1), lambda t: (0, 0)),
        ],
        out_specs=(
            pl.BlockSpec((tb, _E), lambda t: (t, 0)),
            pl.BlockSpec((tb, _E), lambda t: (t, 0)),
        ),
        out_shape=(
            jax.ShapeDtypeStruct((_N, _E), _F32),
            jax.ShapeDtypeStruct((_N, _E), _F32),
        ),
        compiler_params=pltpu.CompilerParams(
            dimension_semantics=("arbitrary",)),
    )(efq_b, kb, vb, cnull)

    tb2 = 1024
    out = pl.pallas_call(
        _mlp_kernel,
        grid=(_N // tb2, _E),
        in_specs=[
            pl.BlockSpec((tb2, _H), lambda t, e: (t, 0)),
            pl.BlockSpec((1, _H, _F), lambda t, e: (e, 0, 0)),
            pl.BlockSpec((1, _H, _F), lambda t, e: (e, 0, 0)),
            pl.BlockSpec((1, _F, _H), lambda t, e: (e, 0, 0)),
            pl.BlockSpec((tb2, _E), lambda t, e: (t, 0)),
        ],
        out_specs=pl.BlockSpec((tb2, _H), lambda t, e: (t, 0)),
        out_shape=jax.ShapeDtypeStruct((_N, _H), _F32),
        compiler_params=pltpu.CompilerParams(
            dimension_semantics=("arbitrary", "arbitrary")),
    )(xb, wg_b, wu_b, wd_b, coeff)

    return out.reshape(b, s, h), w
